# Initial kernel scaffold; baseline (speedup 1.0000x reference)
import jax
import jax.numpy as jnp
from jax.experimental import pallas as pl


def kernel(x, idx_node, kernel, bias):
    raise NotImplementedError("write your pallas kernel here")


# speedup vs baseline: 880.7294x; 880.7294x over previous
import functools

import jax
import jax.numpy as jnp
from jax import lax
from jax.experimental import pallas as pl
from jax.experimental.pallas import tpu as pltpu
from jax.experimental.pallas import tpu_sc as plsc

B, N, P, Q, L = 2, 10000, 16, 16, 8
QL = Q * L
NC, NS = 2, 16
ROWS0 = 5008
ROWS1 = N - ROWS0
CH = 128
FULL_CHUNKS = 39
ACC_ROWS = B * ROWS0
ZR = 632
ZR_LAST = ACC_ROWS - (NS - 1) * ZR
CP = 632
CP_LAST0 = ROWS0 - 7 * CP
CP_LAST1 = ROWS1 - 7 * CP


def _sc_body(xt_hbm, idx_hbm, wt_hbm, bias_hbm, out_hbm,
             x01_v, idx_v, w_v, bias_v, part0_v, part1_v,
             ridx0_v, ridx1_v, ridx0s_v, ridx1s_v, zbuf_v, acc_s):
    c = lax.axis_index("c")
    s = lax.axis_index("s")
    p = s
    base_n = c * ROWS0
    lanes = lax.iota(jnp.int32, 16)
    zeros16 = jnp.zeros((16,), jnp.int32)
    ones16 = jnp.full((16,), 1, jnp.int32)

    pltpu.sync_copy(xt_hbm.at[pl.ds(p, 1)], x01_v)
    pltpu.sync_copy(wt_hbm.at[pl.ds(p, 1)], w_v)
    pltpu.sync_copy(bias_hbm, bias_v)

    @pl.loop(0, zbuf_v.shape[0])
    def _zero(i):
        zbuf_v[i, :] = jnp.zeros((16,), jnp.float32)

    @pl.when(s < NS - 1)
    def _z_full():
        pltpu.sync_copy(zbuf_v, acc_s.at[pl.ds(s * ZR, ZR)])

    @pl.when(s == NS - 1)
    def _z_last():
        pltpu.sync_copy(zbuf_v.at[pl.ds(0, ZR_LAST)],
                        acc_s.at[pl.ds((NS - 1) * ZR, ZR_LAST)])

    plsc.subcore_barrier()

    wvec = [w_v[0, l, :] for l in range(L)]
    pat = [lanes * L + l for l in range(L)]
    biasvec = bias_v[:]
    zf = jnp.zeros((16,), jnp.float32)
    init = jnp.where(jnp.broadcast_to(s == 0, (16,)), biasvec, zf)

    def compute_rows(n0_local, count):
        pltpu.sync_copy(
            idx_hbm.at[pl.ds(base_n + n0_local, count), pl.ds(p * QL, QL)],
            idx_v.at[pl.ds(0, count)])

        @pl.loop(0, count)
        def _node(i):
            nsp = jnp.broadcast_to(i, (16,)).astype(jnp.int32)
            a0 = init
            a1 = init
            for l in range(L):
                iv = plsc.load_gather(idx_v, [nsp, pat[l]])
                x0 = plsc.load_gather(x01_v, [zeros16, zeros16, iv])
                x1 = plsc.load_gather(x01_v, [zeros16, ones16, iv])
                a0 = a0 + wvec[l] * x0
                a1 = a1 + wvec[l] * x1
            part0_v[i, :] = a0
            part1_v[i, :] = a1

    @pl.loop(0, FULL_CHUNKS)
    def _chunk(g):
        n0_local = g * CH
        compute_rows(n0_local, CH)
        for t in range(CH // 16):
            v = jnp.broadcast_to(n0_local + t * 16, (16,)).astype(jnp.int32) + lanes
            ridx0_v[pl.ds(t * 16, 16)] = v
            ridx1_v[pl.ds(t * 16, 16)] = v + ROWS0
        pltpu.sync_copy(part0_v, acc_s.at[ridx0_v], add=True)
        pltpu.sync_copy(part1_v, acc_s.at[ridx1_v], add=True)

    @pl.when(c == 0)
    def _tail():
        n0_local = FULL_CHUNKS * CH
        compute_rows(n0_local, 16)
        v = jnp.broadcast_to(n0_local, (16,)).astype(jnp.int32) + lanes
        ridx0s_v[:] = v
        ridx1s_v[:] = v + ROWS0
        pltpu.sync_copy(part0_v.at[pl.ds(0, 16)], acc_s.at[ridx0s_v], add=True)
        pltpu.sync_copy(part1_v.at[pl.ds(0, 16)], acc_s.at[ridx1s_v], add=True)

    plsc.subcore_barrier()

    b_out = s // (NS // B)
    j = s % (NS // B)
    src0 = b_out * ROWS0 + j * CP
    dst0 = b_out * N + base_n + j * CP

    @pl.when(j < NS // B - 1)
    def _cp_full():
        pltpu.sync_copy(acc_s.at[pl.ds(src0, CP)], out_hbm.at[pl.ds(dst0, CP)])

    @pl.when(jnp.logical_and(c == 0, j == NS // B - 1))
    def _cp_last0():
        pltpu.sync_copy(acc_s.at[pl.ds(src0, CP_LAST0)],
                        out_hbm.at[pl.ds(dst0, CP_LAST0)])

    @pl.when(jnp.logical_and(c == 1, j == NS // B - 1))
    def _cp_last1():
        pltpu.sync_copy(acc_s.at[pl.ds(src0, CP_LAST1)],
                        out_hbm.at[pl.ds(dst0, CP_LAST1)])


@jax.jit
def _lrf_sc(xt, idx2, wt, bias):
    mesh = plsc.VectorSubcoreMesh(core_axis_name="c", subcore_axis_name="s")
    run = pl.kernel(
        _sc_body,
        out_type=jax.ShapeDtypeStruct((B * N, Q), jnp.float32),
        mesh=mesh,
        compiler_params=pltpu.CompilerParams(
            needs_layout_passes=False, use_tc_tiling_on_sc=False),
        scratch_types=[
            pltpu.VMEM((1, B, N), jnp.float32),
            pltpu.VMEM((CH, QL), jnp.int32),
            pltpu.VMEM((1, L, Q), jnp.float32),
            pltpu.VMEM((Q,), jnp.float32),
            pltpu.VMEM((CH, Q), jnp.float32),
            pltpu.VMEM((CH, Q), jnp.float32),
            pltpu.VMEM((CH,), jnp.int32),
            pltpu.VMEM((CH,), jnp.int32),
            pltpu.VMEM((16,), jnp.int32),
            pltpu.VMEM((16,), jnp.int32),
            pltpu.VMEM((ZR, Q), jnp.float32),
            pltpu.VMEM_SHARED((ACC_ROWS, Q), jnp.float32),
        ],
    )
    return run(xt, idx2, wt, bias)


def kernel(x, idx_node, kernel, bias):
    xt = jnp.transpose(x, (2, 0, 1))
    idx2 = idx_node.reshape(N, P * QL)
    wt = jnp.transpose(kernel, (1, 0, 2))
    out = _lrf_sc(xt, idx2, wt, bias)
    return out.reshape(B, N, Q)
